# baseline (speedup 1.0000x reference)
import functools

import numpy as np
import jax
import jax.numpy as jnp
from jax import lax
from jax.experimental import pallas as pl
from jax.experimental.pallas import tpu as pltpu
from jax.experimental.pallas import tpu_sc as plsc

_D = 128
_L = 80
_K = 5
_NEG = 5
_SLOTS = 2 * _K
_NNEG = _L * _SLOTS * _NEG
_NW = 32
_B = 4096
_BPW = _B // _NW
_NDOT = _BPW + _SLOTS * 3
_NDOTP = 160

_NPAIRS = float(sum(min(i + _K, _L - 1) - max(i - _K, 0) for i in range(_L)))


def _static_tables():
    owner = np.zeros(_B, np.int32)
    mmul = np.zeros(_B, np.float32)
    for g in range(_B):
        n = g - _L
        if 0 <= n < _NNEG:
            o = n // (_SLOTS * _NEG)
            slot = (n % (_SLOTS * _NEG)) // _NEG
            w = min(o + _K, _L - 1) - max(o - _K, 0)
            owner[g] = o
            mmul[g] = 1.0 if slot < w else 0.0
    return owner, mmul


_OWNER_NP, _MMUL_NP = _static_tables()


def _softplus16(x):
    y = jnp.exp(-jnp.abs(x))
    t = y / (y + 2.0)
    t2 = t * t
    p = t2 * jnp.float32(1.0 / 9.0) + jnp.float32(1.0 / 7.0)
    p = p * t2 + jnp.float32(1.0 / 5.0)
    p = p * t2 + jnp.float32(1.0 / 3.0)
    p = p * t2 + jnp.float32(1.0)
    return jnp.maximum(x, 0.0) + 2.0 * t * p


def _store1(ref, pos_idx, val):
    lane = lax.iota(jnp.int32, 16)
    idxv = jnp.zeros((16,), jnp.int32) + pos_idx
    valv = jnp.zeros((16,), jnp.float32) + val
    plsc.store_scatter(ref, [idxv], valv, mask=lane == 0)


def _sc_loss(table, mp, neg, owner_t, mmul_t):
    mesh = plsc.VectorSubcoreMesh(core_axis_name="c", subcore_axis_name="s")

    @functools.partial(
        pl.kernel,
        mesh=mesh,
        out_type=jax.ShapeDtypeStruct((_NW, 16), jnp.float32),
        scratch_types=[
            pltpu.VMEM((_L,), jnp.int32),
            pltpu.VMEM((_BPW,), jnp.int32),
            pltpu.VMEM((_L, _D), jnp.float32),
            pltpu.VMEM((_BPW, _D), jnp.float32),
            pltpu.VMEM((_BPW + 16,), jnp.int32),
            pltpu.VMEM((_BPW + 16,), jnp.float32),
            pltpu.VMEM((_NDOTP,), jnp.float32),
            pltpu.VMEM((16,), jnp.float32),
            pltpu.SemaphoreType.DMA,
            pltpu.SemaphoreType.DMA,
        ],
        compiler_params=pltpu.CompilerParams(needs_layout_passes=False),
    )
    def body(table_hbm, mp_hbm, neg_hbm, owner_hbm, mmul_hbm, out_hbm,
             mp_v, idx_v, walk_v, rows_v, owner_v, mmul_v, dots_v, acc_v,
             sem_w, sem_r):
        wid = lax.axis_index("s") * 2 + lax.axis_index("c")
        base = wid * _BPW

        pltpu.sync_copy(mp_hbm, mp_v)
        pltpu.sync_copy(owner_hbm.at[pl.ds(base, _BPW)],
                        owner_v.at[pl.ds(0, _BPW)])
        pltpu.sync_copy(mmul_hbm.at[pl.ds(base, _BPW)],
                        mmul_v.at[pl.ds(0, _BPW)])

        @pl.when(wid == 0)
        def _():
            pltpu.sync_copy(mp_hbm, idx_v.at[pl.ds(0, _L)])
            pltpu.sync_copy(neg_hbm.at[pl.ds(0, _BPW - _L)],
                            idx_v.at[pl.ds(_L, _BPW - _L)])

        @pl.when((wid > 0) & (wid < _NW - 1))
        def _():
            pltpu.sync_copy(neg_hbm.at[pl.ds(base - _L, _BPW)], idx_v)

        @pl.when(wid == _NW - 1)
        def _():
            tail = _NNEG - ((_NW - 1) * _BPW - _L)
            pltpu.sync_copy(neg_hbm.at[pl.ds(_NNEG - tail, tail)],
                            idx_v.at[pl.ds(0, tail)])
            for t in range(tail, _BPW, 16):
                idx_v[pl.ds(t, 16)] = jnp.zeros((16,), jnp.int32)

        cw = pltpu.async_copy(table_hbm.at[mp_v], walk_v, sem_w)
        cr = pltpu.async_copy(table_hbm.at[idx_v], rows_v, sem_r)
        cw.wait()

        dots_v[pl.ds(_BPW, 16)] = jnp.full((16,), -30.0, jnp.float32)
        dots_v[pl.ds(_NDOTP - 16, 16)] = jnp.full((16,), -30.0, jnp.float32)

        for p_i in range(3):
            i = wid + _NW * p_i
            iok = i < _L
            iaddr = jnp.minimum(i, _L - 1)
            wch = [walk_v[iaddr, pl.ds(c * 16, 16)] for c in range(8)]
            koff = 0
            for off in list(range(-_K, 0)) + list(range(1, _K + 1)):
                j = i + off
                jok = iok & (j >= 0) & (j < _L)
                jaddr = jnp.clip(j, 0, _L - 1)
                acc = jnp.zeros((16,), jnp.float32)
                for c in range(8):
                    acc = acc + wch[c] * walk_v[jaddr, pl.ds(c * 16, 16)]
                d = plsc.cumsum(acc)[15]
                m = jok.astype(jnp.float32)
                _store1(dots_v, _BPW + p_i * _SLOTS + koff, m * (30.0 - d) - 30.0)
                koff += 1

        cr.wait()

        def nbody(r, carry):
            o = owner_v[pl.ds(r, 16)][0]
            acc = jnp.zeros((16,), jnp.float32)
            for c in range(8):
                acc = acc + (rows_v[r, pl.ds(c * 16, 16)]
                             * walk_v[o, pl.ds(c * 16, 16)])
            d = plsc.cumsum(acc)[15]
            m = mmul_v[pl.ds(r, 16)][0]
            _store1(dots_v, r, m * (d + 30.0) - 30.0)
            return carry

        lax.fori_loop(0, _BPW, nbody, 0)

        total = jnp.zeros((16,), jnp.float32)
        for c in range(_NDOTP // 16):
            total = total + _softplus16(dots_v[pl.ds(c * 16, 16)])
        acc_v[...] = total
        pltpu.sync_copy(acc_v, out_hbm.at[wid])

    return body(table, mp, neg, owner_t, mmul_t)


def kernel(MP, neg_samples, X):
    mp = MP.astype(jnp.int32)
    neg = neg_samples.astype(jnp.int32).reshape(-1)
    owner_t = jnp.asarray(_OWNER_NP)
    mmul_t = jnp.asarray(_MMUL_NP)
    partials = _sc_loss(X, mp, neg, owner_t, mmul_t)
    return jnp.sum(partials) / jnp.float32(_NPAIRS)
